# baseline (speedup 1.0000x reference)
import functools

import jax
import jax.numpy as jnp
from jax import lax
from jax.experimental import pallas as pl
from jax.experimental.pallas import tpu as pltpu
from jax.experimental.pallas import tpu_sc as plsc

N = 10000
E = 320000
H = 8
D = 128
DH = 16
ACCW = 144
NPAD = 10240
NC = 2
NS = 16
NW = NC * NS
EPT = E // NW
CH = 40
NCHUNK = EPT // CH
NPAIR = NCHUNK // 2
RPT = NPAD // NS
RCH = CH
NRCH = RPT // RCH
BN = 400
BNC = 80
A1OFF = NPAD // BNC


def _edge_body(featx_hbm, er_hbm, src_hbm, dst_hbm, w_hbm, out_hbm,
               src_a, dst_a, w_a, sdst_a, er_a, featx_a, msg_a,
               src_b, dst_b, w_b, sdst_b, er_b, featx_b, msg_b,
               acc_sh, semi_a, semg_a, sems_a, semi_b, semg_b, sems_b):
    cid = lax.axis_index("c")
    sid = lax.axis_index("s")
    wid = cid * NS + sid

    @plsc.parallel_loop(0, RCH, unroll=4)
    def _zrow(i):
        for j in range(ACCW // 16):
            msg_a[i, pl.ds(j * 16, 16)] = jnp.zeros((16,), jnp.float32)
    row0 = sid * RPT

    def _zcopy(k, carry):
        pltpu.sync_copy(msg_a, acc_sh.at[pl.ds(row0 + k * RCH, RCH)])
        return carry
    lax.fori_loop(0, NRCH, _zcopy, 0)
    plsc.subcore_barrier()

    lanes = lax.iota(jnp.int32, 16)
    headmask = lanes < 8
    zero16 = jnp.zeros((16,), jnp.float32)
    ebase = wid * EPT

    def _idx_copies(ci, src_v, dst_v, w_v, sem):
        b = ebase + ci * CH
        c1 = pltpu.async_copy(src_hbm.at[pl.ds(b, CH)], src_v, sem)
        c2 = pltpu.async_copy(dst_hbm.at[pl.ds(b, CH)], dst_v, sem)
        c3 = pltpu.async_copy(w_hbm.at[pl.ds(b, CH)], w_v.at[pl.ds(0, CH)],
                              sem)
        return c1, c2, c3

    def _gathers(src_v, dst_v, er_v, featx_v, sem):
        g1 = pltpu.async_copy(er_hbm.at[dst_v], er_v, sem)
        g2 = pltpu.async_copy(featx_hbm.at[src_v], featx_v, sem)
        return g1, g2

    def _save_dst(dst_v, sdst_v):
        for j in (0, 16, CH - 16):
            sdst_v[pl.ds(j, 16)] = dst_v[pl.ds(j, 16)]

    def _stage1(featx_v, er_v, w_v, msg_v):
        @plsc.parallel_loop(0, CH, unroll=8)
        def _s1(c):
            ev = featx_v[c, pl.ds(D, 16)] + er_v[c, :]
            ev = jnp.where(ev > 0, ev, 0.1 * ev)
            wv = w_v[pl.ds(c, 16)]
            msg_v[c, pl.ds(D, 16)] = ev * wv[0]

    def _stage2(featx_v, msg_v):
        @plsc.parallel_loop(0, CH, unroll=4)
        def _s2(c):
            ex = jnp.exp(msg_v[c, pl.ds(D, 16)])
            msg_v[c, pl.ds(D, 16)] = jnp.where(headmask, ex, zero16)
            for h in range(H):
                msg_v[c, pl.ds(h * DH, DH)] = featx_v[c, pl.ds(h * DH, DH)] * ex[h]

    def _wait_gathers(src_v, dst_v, er_v, featx_v, sem):
        pltpu.make_async_copy(er_hbm.at[dst_v], er_v, sem).wait()
        pltpu.make_async_copy(featx_hbm.at[src_v], featx_v, sem).wait()

    def _phase(nci, src_v, dst_v, w_v, sdst_v, er_v, featx_v, msg_v,
               semi, semg, sems):
        _wait_gathers(src_v, dst_v, er_v, featx_v, semg)
        _save_dst(dst_v, sdst_v)
        _stage1(featx_v, er_v, w_v, msg_v)
        ih = None
        if nci is not None:
            ih = _idx_copies(nci, src_v, dst_v, w_v, semi)
        _stage2(featx_v, msg_v)
        if nci is not None:
            for c in ih:
                c.wait()
            _gathers(src_v, dst_v, er_v, featx_v, semg)
        return pltpu.async_copy(msg_v, acc_sh.at[sdst_v], sems, add=True)

    for c in _idx_copies(0, src_a, dst_a, w_a, semi_a):
        c.wait()
    _gathers(src_a, dst_a, er_a, featx_a, semg_a)
    for c in _idx_copies(1, src_b, dst_b, w_b, semi_b):
        c.wait()
    _gathers(src_b, dst_b, er_b, featx_b, semg_b)

    def _pair(k, carry):
        sh_a = _phase(2 * k + 2, src_a, dst_a, w_a, sdst_a,
                      er_a, featx_a, msg_a, semi_a, semg_a, sems_a)
        sh_b = _phase(2 * k + 3, src_b, dst_b, w_b, sdst_b,
                      er_b, featx_b, msg_b, semi_b, semg_b, sems_b)
        sh_a.wait()
        sh_b.wait()
        return carry

    lax.fori_loop(0, NPAIR - 1, _pair, 0)

    sh_a = _phase(None, src_a, dst_a, w_a, sdst_a,
                  er_a, featx_a, msg_a, semi_a, semg_a, sems_a)
    sh_b = _phase(None, src_b, dst_b, w_b, sdst_b,
                  er_b, featx_b, msg_b, semi_b, semg_b, sems_b)
    sh_a.wait()
    sh_b.wait()
    plsc.subcore_barrier()

    def _ocopy(k, carry):
        r = row0 + k * RCH
        pltpu.sync_copy(acc_sh.at[pl.ds(r, RCH)], msg_a)
        pltpu.sync_copy(msg_a, out_hbm.at[pl.ds(cid * NPAD + r, RCH)])
        return carry
    lax.fori_loop(0, NRCH, _ocopy, 0)


@functools.cache
def _edge_call():
    return pl.kernel(
        _edge_body,
        out_type=jax.ShapeDtypeStruct((NC * NPAD, ACCW), jnp.float32),
        mesh=plsc.VectorSubcoreMesh(core_axis_name="c", subcore_axis_name="s",
                                    num_cores=NC, num_subcores=NS),
        scratch_types=[
            pltpu.VMEM((CH,), jnp.int32),
            pltpu.VMEM((CH,), jnp.int32),
            pltpu.VMEM((CH + 16,), jnp.float32),
            pltpu.VMEM((CH,), jnp.int32),
            pltpu.VMEM((CH, 16), jnp.float32),
            pltpu.VMEM((CH, ACCW), jnp.float32),
            pltpu.VMEM((CH, ACCW), jnp.float32),
            pltpu.VMEM((CH,), jnp.int32),
            pltpu.VMEM((CH,), jnp.int32),
            pltpu.VMEM((CH + 16,), jnp.float32),
            pltpu.VMEM((CH,), jnp.int32),
            pltpu.VMEM((CH, 16), jnp.float32),
            pltpu.VMEM((CH, ACCW), jnp.float32),
            pltpu.VMEM((CH, ACCW), jnp.float32),
            pltpu.VMEM_SHARED((NPAD, ACCW), jnp.float32),
            pltpu.SemaphoreType.DMA,
            pltpu.SemaphoreType.DMA,
            pltpu.SemaphoreType.DMA,
            pltpu.SemaphoreType.DMA,
            pltpu.SemaphoreType.DMA,
            pltpu.SemaphoreType.DMA,
        ],
        compiler_params=pltpu.CompilerParams(use_tc_tiling_on_sc=False,
                                             internal_scratch_in_bytes=2 ** 17),
    )


def _feat_ext(x, w, al, ar):
    feat = lax.dot_general(x, w, (((1,), (1,)), ((), ())),
                           preferred_element_type=jnp.float32)
    el = jnp.dot(feat, al, preferred_element_type=jnp.float32)
    er = jnp.dot(feat, ar, preferred_element_type=jnp.float32)
    return jnp.concatenate([feat, el], axis=1), er


def _feat_body(x_ref, w_ref, al_ref, ar_ref, fx_ref, er_ref):
    fx, er = _feat_ext(x_ref[...], w_ref[...], al_ref[...], ar_ref[...])
    fx_ref[...] = fx
    er_ref[...] = er


_feat_call = pl.pallas_call(
    _feat_body,
    grid=(N // BN,),
    in_specs=[pl.BlockSpec((BN, D), lambda i: (i, 0)),
              pl.BlockSpec((D, D), lambda i: (0, 0)),
              pl.BlockSpec((D, 16), lambda i: (0, 0)),
              pl.BlockSpec((D, 16), lambda i: (0, 0))],
    out_specs=[pl.BlockSpec((BN, ACCW), lambda i: (i, 0)),
               pl.BlockSpec((BN, 16), lambda i: (i, 0))],
    out_shape=[jax.ShapeDtypeStruct((N, ACCW), jnp.float32),
               jax.ShapeDtypeStruct((N, 16), jnp.float32)],
)


def _combine_rst(a0, a1, r, bd, b):
    a = a0 + a1
    den = jnp.dot(a[:, D:D + H], r, preferred_element_type=jnp.float32)
    rst = jnp.where(den > 0, a[:, :D] / den, 0.0)
    y = jnp.dot(rst, bd, preferred_element_type=jnp.float32) + b
    return jnp.maximum(y, 0.0)


def _combine_feat_body(a0_ref, a1_ref, r_ref, bd_ref, b_ref, w_ref, al_ref,
                       ar_ref, x1_ref, fx_ref, er_ref):
    x1 = _combine_rst(a0_ref[...], a1_ref[...], r_ref[...], bd_ref[...],
                      b_ref[...])
    x1_ref[...] = x1
    fx, er = _feat_ext(x1, w_ref[...], al_ref[...], ar_ref[...])
    fx_ref[...] = fx
    er_ref[...] = er


_combine_feat_call = pl.pallas_call(
    _combine_feat_body,
    grid=(N // BNC,),
    in_specs=[pl.BlockSpec((BNC, ACCW), lambda i: (i, 0)),
              pl.BlockSpec((BNC, ACCW), lambda i: (A1OFF + i, 0)),
              pl.BlockSpec((H, D), lambda i: (0, 0)),
              pl.BlockSpec((D, D), lambda i: (0, 0)),
              pl.BlockSpec((1, D), lambda i: (0, 0)),
              pl.BlockSpec((D, D), lambda i: (0, 0)),
              pl.BlockSpec((D, 16), lambda i: (0, 0)),
              pl.BlockSpec((D, 16), lambda i: (0, 0))],
    out_specs=[pl.BlockSpec((BNC, D), lambda i: (i, 0)),
               pl.BlockSpec((BNC, ACCW), lambda i: (i, 0)),
               pl.BlockSpec((BNC, 16), lambda i: (i, 0))],
    out_shape=[jax.ShapeDtypeStruct((N, D), jnp.float32),
               jax.ShapeDtypeStruct((N, ACCW), jnp.float32),
               jax.ShapeDtypeStruct((N, 16), jnp.float32)],
)


def _final_body(a0_ref, a1_ref, r_ref, bd_ref, b_ref, x1_ref, w0_ref, w1_ref,
                mb_ref, o_ref):
    out1 = _combine_rst(a0_ref[...], a1_ref[...], r_ref[...], bd_ref[...],
                        b_ref[...])
    o = (jnp.dot(x1_ref[...], w0_ref[...], preferred_element_type=jnp.float32)
         + jnp.dot(out1, w1_ref[...], preferred_element_type=jnp.float32)
         + mb_ref[...])
    o_ref[...] = o


_final_call = pl.pallas_call(
    _final_body,
    grid=(N // BNC,),
    in_specs=[pl.BlockSpec((BNC, ACCW), lambda i: (i, 0)),
              pl.BlockSpec((BNC, ACCW), lambda i: (A1OFF + i, 0)),
              pl.BlockSpec((H, D), lambda i: (0, 0)),
              pl.BlockSpec((D, D), lambda i: (0, 0)),
              pl.BlockSpec((1, D), lambda i: (0, 0)),
              pl.BlockSpec((BNC, D), lambda i: (i, 0)),
              pl.BlockSpec((D, D), lambda i: (0, 0)),
              pl.BlockSpec((D, D), lambda i: (0, 0)),
              pl.BlockSpec((1, D), lambda i: (0, 0))],
    out_specs=pl.BlockSpec((BNC, D), lambda i: (i, 0)),
    out_shape=jax.ShapeDtypeStruct((N, D), jnp.float32),
)


def _attn_mats(attn_l, attn_r):
    m = jnp.kron(jnp.eye(H, dtype=jnp.float32),
                 jnp.ones((DH, 1), jnp.float32))
    al = attn_l.reshape(H * DH)[:, None] * m
    ar = attn_r.reshape(H * DH)[:, None] * m
    return (jnp.concatenate([al, al], axis=1),
            jnp.concatenate([ar, ar], axis=1))


def kernel(solutions, edge_index, edge_w, fc_W0, attn_l0, attn_r0, out_W0,
           out_b0, fc_W1, attn_l1, attn_r1, out_W1, out_b1, mlp_W, mlp_b):
    src = edge_index[0]
    dst = edge_index[1]
    x0 = jnp.concatenate([solutions[0], solutions[1]], axis=-1)[0]

    r_mat = jnp.kron(jnp.eye(H, dtype=jnp.float32),
                     jnp.ones((1, DH), jnp.float32))
    bd0 = jnp.kron(jnp.eye(H, dtype=jnp.float32), out_W0.T)
    bd1 = jnp.kron(jnp.eye(H, dtype=jnp.float32), out_W1.T)
    b0t = jnp.tile(out_b0, H)[None, :]
    b1t = jnp.tile(out_b1, H)[None, :]
    w0t = mlp_W[:, :D].T
    w1t = mlp_W[:, D:].T
    mbt = mlp_b[None, :]

    al0, ar0 = _attn_mats(attn_l0, attn_r0)
    al1, ar1 = _attn_mats(attn_l1, attn_r1)
    featx0, er0 = _feat_call(x0, fc_W0, al0, ar0)
    acc0 = _edge_call()(featx0, er0, src, dst, edge_w)
    x1, featx1, er1 = _combine_feat_call(acc0, acc0, r_mat, bd0, b0t,
                                         fc_W1, al1, ar1)
    acc1 = _edge_call()(featx1, er1, src, dst, edge_w)
    h = _final_call(acc1, acc1, r_mat, bd1, b1t, x1, w0t, w1t, mbt)
    return h[None]


# speedup vs baseline: 1.1564x; 1.1564x over previous
import functools

import jax
import jax.numpy as jnp
from jax import lax
from jax.experimental import pallas as pl
from jax.experimental.pallas import tpu as pltpu
from jax.experimental.pallas import tpu_sc as plsc

N = 10000
E = 320000
H = 8
D = 128
DH = 16
ACCW = 144
NPAD = 10240
NC = 2
NS = 16
NW = NC * NS
EPT = E // NW
CH = 40
NCHUNK = EPT // CH
NPAIR = NCHUNK // 2
RPT = NPAD // NS
RCH = CH
NRCH = RPT // RCH
BN = 400


def _edge_body(featx_hbm, er_hbm, src_hbm, dst_hbm, w_hbm, out_hbm,
               src_a, dst_a, w_a, sdst_a, er_a, featx_a, msg_a,
               src_b, dst_b, w_b, sdst_b, er_b, featx_b, msg_b,
               acc_sh, semi_a, semg_a, sems_a, semi_b, semg_b, sems_b):
    cid = lax.axis_index("c")
    sid = lax.axis_index("s")
    wid = cid * NS + sid

    @plsc.parallel_loop(0, RCH, unroll=4)
    def _zrow(i):
        for j in range(ACCW // 16):
            msg_a[i, pl.ds(j * 16, 16)] = jnp.zeros((16,), jnp.float32)
    row0 = sid * RPT

    def _zcopy(k, carry):
        pltpu.sync_copy(msg_a, acc_sh.at[pl.ds(row0 + k * RCH, RCH)])
        return carry
    lax.fori_loop(0, NRCH, _zcopy, 0)
    plsc.subcore_barrier()

    lanes = lax.iota(jnp.int32, 16)
    headmask = lanes < 8
    zero16 = jnp.zeros((16,), jnp.float32)
    ebase = wid * EPT

    def _idx_copies(ci, src_v, dst_v, w_v, sem):
        b = ebase + ci * CH
        c1 = pltpu.async_copy(src_hbm.at[pl.ds(b, CH)], src_v, sem)
        c2 = pltpu.async_copy(dst_hbm.at[pl.ds(b, CH)], dst_v, sem)
        c3 = pltpu.async_copy(w_hbm.at[pl.ds(b, CH)], w_v.at[pl.ds(0, CH)],
                              sem)
        return c1, c2, c3

    def _gathers(src_v, dst_v, er_v, featx_v, sem):
        g1 = pltpu.async_copy(er_hbm.at[dst_v], er_v, sem)
        g2 = pltpu.async_copy(featx_hbm.at[src_v], featx_v, sem)
        return g1, g2

    def _save_dst(dst_v, sdst_v):
        for j in (0, 16, CH - 16):
            sdst_v[pl.ds(j, 16)] = dst_v[pl.ds(j, 16)]

    def _stage1(featx_v, er_v, w_v, msg_v):
        @plsc.parallel_loop(0, CH, unroll=8)
        def _s1(c):
            ev = featx_v[c, pl.ds(D, 16)] + er_v[c, :]
            ev = jnp.where(ev > 0, ev, 0.1 * ev)
            wv = w_v[pl.ds(c, 16)]
            msg_v[c, pl.ds(D, 16)] = ev * wv[0]

    def _stage2(featx_v, msg_v):
        @plsc.parallel_loop(0, CH, unroll=4)
        def _s2(c):
            ex = jnp.exp(msg_v[c, pl.ds(D, 16)])
            msg_v[c, pl.ds(D, 16)] = jnp.where(headmask, ex, zero16)
            for h in range(H):
                msg_v[c, pl.ds(h * DH, DH)] = featx_v[c, pl.ds(h * DH, DH)] * ex[h]

    def _wait_gathers(src_v, dst_v, er_v, featx_v, sem):
        pltpu.make_async_copy(er_hbm.at[dst_v], er_v, sem).wait()
        pltpu.make_async_copy(featx_hbm.at[src_v], featx_v, sem).wait()

    def _phase(nci, src_v, dst_v, w_v, sdst_v, er_v, featx_v, msg_v,
               semi, semg, sems):
        _wait_gathers(src_v, dst_v, er_v, featx_v, semg)
        _save_dst(dst_v, sdst_v)
        _stage1(featx_v, er_v, w_v, msg_v)
        ih = None
        if nci is not None:
            ih = _idx_copies(nci, src_v, dst_v, w_v, semi)
        _stage2(featx_v, msg_v)
        if nci is not None:
            for c in ih:
                c.wait()
            _gathers(src_v, dst_v, er_v, featx_v, semg)
        return pltpu.async_copy(msg_v, acc_sh.at[sdst_v], sems, add=True)

    for c in _idx_copies(0, src_a, dst_a, w_a, semi_a):
        c.wait()
    _gathers(src_a, dst_a, er_a, featx_a, semg_a)
    for c in _idx_copies(1, src_b, dst_b, w_b, semi_b):
        c.wait()
    _gathers(src_b, dst_b, er_b, featx_b, semg_b)

    def _pair(k, carry):
        sh_a = _phase(2 * k + 2, src_a, dst_a, w_a, sdst_a,
                      er_a, featx_a, msg_a, semi_a, semg_a, sems_a)
        sh_b = _phase(2 * k + 3, src_b, dst_b, w_b, sdst_b,
                      er_b, featx_b, msg_b, semi_b, semg_b, sems_b)
        sh_a.wait()
        sh_b.wait()
        return carry

    lax.fori_loop(0, NPAIR - 1, _pair, 0)

    sh_a = _phase(None, src_a, dst_a, w_a, sdst_a,
                  er_a, featx_a, msg_a, semi_a, semg_a, sems_a)
    sh_b = _phase(None, src_b, dst_b, w_b, sdst_b,
                  er_b, featx_b, msg_b, semi_b, semg_b, sems_b)
    sh_a.wait()
    sh_b.wait()
    plsc.subcore_barrier()

    def _ocopy(k, carry):
        r = row0 + k * RCH
        pltpu.sync_copy(acc_sh.at[pl.ds(r, RCH)], msg_a)
        pltpu.sync_copy(msg_a, out_hbm.at[pl.ds(cid * NPAD + r, RCH)])
        return carry
    lax.fori_loop(0, NRCH, _ocopy, 0)


@functools.cache
def _edge_call():
    return pl.kernel(
        _edge_body,
        out_type=jax.ShapeDtypeStruct((NC * NPAD, ACCW), jnp.float32),
        mesh=plsc.VectorSubcoreMesh(core_axis_name="c", subcore_axis_name="s",
                                    num_cores=NC, num_subcores=NS),
        scratch_types=[
            pltpu.VMEM((CH,), jnp.int32),
            pltpu.VMEM((CH,), jnp.int32),
            pltpu.VMEM((CH + 16,), jnp.float32),
            pltpu.VMEM((CH,), jnp.int32),
            pltpu.VMEM((CH, 16), jnp.float32),
            pltpu.VMEM((CH, ACCW), jnp.float32),
            pltpu.VMEM((CH, ACCW), jnp.float32),
            pltpu.VMEM((CH,), jnp.int32),
            pltpu.VMEM((CH,), jnp.int32),
            pltpu.VMEM((CH + 16,), jnp.float32),
            pltpu.VMEM((CH,), jnp.int32),
            pltpu.VMEM((CH, 16), jnp.float32),
            pltpu.VMEM((CH, ACCW), jnp.float32),
            pltpu.VMEM((CH, ACCW), jnp.float32),
            pltpu.VMEM_SHARED((NPAD, ACCW), jnp.float32),
            pltpu.SemaphoreType.DMA,
            pltpu.SemaphoreType.DMA,
            pltpu.SemaphoreType.DMA,
            pltpu.SemaphoreType.DMA,
            pltpu.SemaphoreType.DMA,
            pltpu.SemaphoreType.DMA,
        ],
        compiler_params=pltpu.CompilerParams(use_tc_tiling_on_sc=False,
                                             internal_scratch_in_bytes=2 ** 17),
    )


def _feat_ext(x, w, al, ar):
    feat = lax.dot_general(x, w, (((1,), (1,)), ((), ())),
                           preferred_element_type=jnp.float32)
    el = jnp.dot(feat, al, preferred_element_type=jnp.float32)
    er = jnp.dot(feat, ar, preferred_element_type=jnp.float32)
    return jnp.concatenate([feat, el], axis=1), er


def _feat_body(x_ref, w_ref, al_ref, ar_ref, fx_ref, er_ref):
    fx, er = _feat_ext(x_ref[...], w_ref[...], al_ref[...], ar_ref[...])
    fx_ref[...] = fx
    er_ref[...] = er


_feat_call = pl.pallas_call(
    _feat_body,
    grid=(N // BN,),
    in_specs=[pl.BlockSpec((BN, D), lambda i: (i, 0)),
              pl.BlockSpec((D, D), lambda i: (0, 0)),
              pl.BlockSpec((D, 16), lambda i: (0, 0)),
              pl.BlockSpec((D, 16), lambda i: (0, 0))],
    out_specs=[pl.BlockSpec((BN, ACCW), lambda i: (i, 0)),
               pl.BlockSpec((BN, 16), lambda i: (i, 0))],
    out_shape=[jax.ShapeDtypeStruct((N, ACCW), jnp.float32),
               jax.ShapeDtypeStruct((N, 16), jnp.float32)],
)


def _combine_rst(a0, a1, r, bd, b):
    a = a0 + a1
    den = jnp.dot(a[:, D:D + H], r, preferred_element_type=jnp.float32)
    rst = jnp.where(den > 0, a[:, :D] / den, 0.0)
    y = jnp.dot(rst, bd, preferred_element_type=jnp.float32) + b
    return jnp.maximum(y, 0.0)


def _combine_feat_body(a0_ref, a1_ref, r_ref, bd_ref, b_ref, w_ref, al_ref,
                       ar_ref, x1_ref, fx_ref, er_ref):
    x1 = _combine_rst(a0_ref[...], a1_ref[...], r_ref[...], bd_ref[...],
                      b_ref[...])
    x1_ref[...] = x1
    fx, er = _feat_ext(x1, w_ref[...], al_ref[...], ar_ref[...])
    fx_ref[...] = fx
    er_ref[...] = er


_combine_feat_call = pl.pallas_call(
    _combine_feat_body,
    grid=(N // BN,),
    in_specs=[pl.BlockSpec((BN, ACCW), lambda i: (i, 0)),
              pl.BlockSpec((BN, ACCW), lambda i: (i, 0)),
              pl.BlockSpec((H, D), lambda i: (0, 0)),
              pl.BlockSpec((D, D), lambda i: (0, 0)),
              pl.BlockSpec((1, D), lambda i: (0, 0)),
              pl.BlockSpec((D, D), lambda i: (0, 0)),
              pl.BlockSpec((D, 16), lambda i: (0, 0)),
              pl.BlockSpec((D, 16), lambda i: (0, 0))],
    out_specs=[pl.BlockSpec((BN, D), lambda i: (i, 0)),
               pl.BlockSpec((BN, ACCW), lambda i: (i, 0)),
               pl.BlockSpec((BN, 16), lambda i: (i, 0))],
    out_shape=[jax.ShapeDtypeStruct((N, D), jnp.float32),
               jax.ShapeDtypeStruct((N, ACCW), jnp.float32),
               jax.ShapeDtypeStruct((N, 16), jnp.float32)],
)


def _final_body(a0_ref, a1_ref, r_ref, bd_ref, b_ref, x1_ref, w0_ref, w1_ref,
                mb_ref, o_ref):
    out1 = _combine_rst(a0_ref[...], a1_ref[...], r_ref[...], bd_ref[...],
                        b_ref[...])
    o = (jnp.dot(x1_ref[...], w0_ref[...], preferred_element_type=jnp.float32)
         + jnp.dot(out1, w1_ref[...], preferred_element_type=jnp.float32)
         + mb_ref[...])
    o_ref[...] = o


_final_call = pl.pallas_call(
    _final_body,
    grid=(N // BN,),
    in_specs=[pl.BlockSpec((BN, ACCW), lambda i: (i, 0)),
              pl.BlockSpec((BN, ACCW), lambda i: (i, 0)),
              pl.BlockSpec((H, D), lambda i: (0, 0)),
              pl.BlockSpec((D, D), lambda i: (0, 0)),
              pl.BlockSpec((1, D), lambda i: (0, 0)),
              pl.BlockSpec((BN, D), lambda i: (i, 0)),
              pl.BlockSpec((D, D), lambda i: (0, 0)),
              pl.BlockSpec((D, D), lambda i: (0, 0)),
              pl.BlockSpec((1, D), lambda i: (0, 0))],
    out_specs=pl.BlockSpec((BN, D), lambda i: (i, 0)),
    out_shape=jax.ShapeDtypeStruct((N, D), jnp.float32),
)


def _attn_mats(attn_l, attn_r):
    m = jnp.kron(jnp.eye(H, dtype=jnp.float32),
                 jnp.ones((DH, 1), jnp.float32))
    al = attn_l.reshape(H * DH)[:, None] * m
    ar = attn_r.reshape(H * DH)[:, None] * m
    return (jnp.concatenate([al, al], axis=1),
            jnp.concatenate([ar, ar], axis=1))


def kernel(solutions, edge_index, edge_w, fc_W0, attn_l0, attn_r0, out_W0,
           out_b0, fc_W1, attn_l1, attn_r1, out_W1, out_b1, mlp_W, mlp_b):
    src = edge_index[0]
    dst = edge_index[1]
    x0 = jnp.concatenate([solutions[0], solutions[1]], axis=-1)[0]

    r_mat = jnp.kron(jnp.eye(H, dtype=jnp.float32),
                     jnp.ones((1, DH), jnp.float32))
    bd0 = jnp.kron(jnp.eye(H, dtype=jnp.float32), out_W0.T)
    bd1 = jnp.kron(jnp.eye(H, dtype=jnp.float32), out_W1.T)
    b0t = jnp.tile(out_b0, H)[None, :]
    b1t = jnp.tile(out_b1, H)[None, :]
    w0t = mlp_W[:, :D].T
    w1t = mlp_W[:, D:].T
    mbt = mlp_b[None, :]

    al0, ar0 = _attn_mats(attn_l0, attn_r0)
    al1, ar1 = _attn_mats(attn_l1, attn_r1)
    featx0, er0 = _feat_call(x0, fc_W0, al0, ar0)
    acc0 = _edge_call()(featx0, er0, src, dst, edge_w)
    x1, featx1, er1 = _combine_feat_call(acc0[:N], acc0[NPAD:NPAD + N],
                                         r_mat, bd0, b0t, fc_W1, al1, ar1)
    acc1 = _edge_call()(featx1, er1, src, dst, edge_w)
    h = _final_call(acc1[:N], acc1[NPAD:NPAD + N], r_mat, bd1, b1t, x1,
                    w0t, w1t, mbt)
    return h[None]


# speedup vs baseline: 1.2019x; 1.0393x over previous
import functools

import jax
import jax.numpy as jnp
from jax import lax
from jax.experimental import pallas as pl
from jax.experimental.pallas import tpu as pltpu
from jax.experimental.pallas import tpu_sc as plsc

N = 10000
E = 320000
H = 8
D = 128
DH = 16
ACCW = 144
NPAD = 10240
NC = 2
NS = 16
NW = NC * NS
EPT = E // NW
CH = 40
NCHUNK = EPT // CH
NPAIR = NCHUNK // 2
RPT = NPAD // NS
RCH = CH
NRCH = RPT // RCH
BN = 400


def _edge_body(feat_hbm, el_hbm, er_hbm, src_hbm, dst_hbm, w_hbm,
               num_hbm, den_hbm,
               src_a, dst_a, w_a, sdst_a, el_a, er_a, feat_a, msg_a,
               src_b, dst_b, w_b, sdst_b, el_b, er_b, feat_b, msg_b,
               acc_sh, semi_a, semg_a, sems_a, semi_b, semg_b, sems_b):
    cid = lax.axis_index("c")
    sid = lax.axis_index("s")
    wid = cid * NS + sid

    @plsc.parallel_loop(0, RCH, unroll=4)
    def _zrow(i):
        for j in range(ACCW // 16):
            msg_a[i, pl.ds(j * 16, 16)] = jnp.zeros((16,), jnp.float32)
    row0 = sid * RPT

    def _zcopy(k, carry):
        pltpu.sync_copy(msg_a, acc_sh.at[pl.ds(row0 + k * RCH, RCH)])
        return carry
    lax.fori_loop(0, NRCH, _zcopy, 0)
    plsc.subcore_barrier()

    lanes = lax.iota(jnp.int32, 16)
    headmask = lanes < 8
    zero16 = jnp.zeros((16,), jnp.float32)
    ebase = wid * EPT

    def _idx_copies(ci, src_v, dst_v, w_v, sem):
        b = ebase + ci * CH
        c1 = pltpu.async_copy(src_hbm.at[pl.ds(b, CH)], src_v, sem)
        c2 = pltpu.async_copy(dst_hbm.at[pl.ds(b, CH)], dst_v, sem)
        c3 = pltpu.async_copy(w_hbm.at[pl.ds(b, CH)], w_v.at[pl.ds(0, CH)],
                              sem)
        return c1, c2, c3

    def _gathers(src_v, dst_v, el_v, er_v, feat_v, sem):
        g1 = pltpu.async_copy(el_hbm.at[src_v], el_v, sem)
        g2 = pltpu.async_copy(er_hbm.at[dst_v], er_v, sem)
        g3 = pltpu.async_copy(feat_hbm.at[src_v], feat_v, sem)
        return g1, g2, g3

    def _save_dst(dst_v, sdst_v):
        for j in (0, 16, CH - 16):
            sdst_v[pl.ds(j, 16)] = dst_v[pl.ds(j, 16)]

    def _stage1(el_v, er_v, w_v, msg_v):
        @plsc.parallel_loop(0, CH, unroll=8)
        def _s1(c):
            ev = el_v[c, :] + er_v[c, :]
            ev = jnp.where(ev > 0, ev, 0.1 * ev)
            wv = w_v[pl.ds(c, 16)]
            msg_v[c, pl.ds(D, 16)] = ev * wv[0]

    def _stage2(feat_v, msg_v):
        @plsc.parallel_loop(0, CH, unroll=4)
        def _s2(c):
            ex = jnp.exp(msg_v[c, pl.ds(D, 16)])
            msg_v[c, pl.ds(D, 16)] = jnp.where(headmask, ex, zero16)
            for h in range(H):
                msg_v[c, pl.ds(h * DH, DH)] = feat_v[c, pl.ds(h * DH, DH)] * ex[h]

    def _wait_gathers(src_v, dst_v, el_v, er_v, feat_v, sem):
        pltpu.make_async_copy(el_hbm.at[src_v], el_v, sem).wait()
        pltpu.make_async_copy(er_hbm.at[dst_v], er_v, sem).wait()
        pltpu.make_async_copy(feat_hbm.at[src_v], feat_v, sem).wait()

    def _phase(nci, src_v, dst_v, w_v, sdst_v, el_v, er_v, feat_v, msg_v,
               semi, semg, sems):
        _wait_gathers(src_v, dst_v, el_v, er_v, feat_v, semg)
        _save_dst(dst_v, sdst_v)
        _stage1(el_v, er_v, w_v, msg_v)
        ih = None
        if nci is not None:
            ih = _idx_copies(nci, src_v, dst_v, w_v, semi)
        _stage2(feat_v, msg_v)
        if nci is not None:
            for c in ih:
                c.wait()
            _gathers(src_v, dst_v, el_v, er_v, feat_v, semg)
        return pltpu.async_copy(msg_v, acc_sh.at[sdst_v], sems, add=True)

    for c in _idx_copies(0, src_a, dst_a, w_a, semi_a):
        c.wait()
    _gathers(src_a, dst_a, el_a, er_a, feat_a, semg_a)
    for c in _idx_copies(1, src_b, dst_b, w_b, semi_b):
        c.wait()
    _gathers(src_b, dst_b, el_b, er_b, feat_b, semg_b)

    def _pair(k, carry):
        sh_a = _phase(2 * k + 2, src_a, dst_a, w_a, sdst_a,
                      el_a, er_a, feat_a, msg_a, semi_a, semg_a, sems_a)
        sh_b = _phase(2 * k + 3, src_b, dst_b, w_b, sdst_b,
                      el_b, er_b, feat_b, msg_b, semi_b, semg_b, sems_b)
        sh_a.wait()
        sh_b.wait()
        return carry

    lax.fori_loop(0, NPAIR - 1, _pair, 0)

    sh_a = _phase(None, src_a, dst_a, w_a, sdst_a,
                  el_a, er_a, feat_a, msg_a, semi_a, semg_a, sems_a)
    sh_b = _phase(None, src_b, dst_b, w_b, sdst_b,
                  el_b, er_b, feat_b, msg_b, semi_b, semg_b, sems_b)
    sh_a.wait()
    sh_b.wait()
    plsc.subcore_barrier()

    def _ocopy(k, carry):
        r = row0 + k * RCH
        pltpu.sync_copy(acc_sh.at[pl.ds(r, RCH)], msg_a)

        @plsc.parallel_loop(0, RCH, unroll=4)
        def _split(i):
            for j in range(D // 16):
                feat_a[i, pl.ds(j * 16, 16)] = msg_a[i, pl.ds(j * 16, 16)]
            el_a[i, :] = msg_a[i, pl.ds(D, 16)]
        pltpu.sync_copy(feat_a, num_hbm.at[pl.ds(cid * NPAD + r, RCH)])
        pltpu.sync_copy(el_a, den_hbm.at[pl.ds(cid * NPAD + r, RCH)])
        return carry
    lax.fori_loop(0, NRCH, _ocopy, 0)


@functools.cache
def _edge_call():
    return pl.kernel(
        _edge_body,
        out_type=[jax.ShapeDtypeStruct((NC * NPAD, D), jnp.float32),
                  jax.ShapeDtypeStruct((NC * NPAD, 16), jnp.float32)],
        mesh=plsc.VectorSubcoreMesh(core_axis_name="c", subcore_axis_name="s",
                                    num_cores=NC, num_subcores=NS),
        scratch_types=[
            pltpu.VMEM((CH,), jnp.int32),
            pltpu.VMEM((CH,), jnp.int32),
            pltpu.VMEM((CH + 16,), jnp.float32),
            pltpu.VMEM((CH,), jnp.int32),
            pltpu.VMEM((CH, 16), jnp.float32),
            pltpu.VMEM((CH, 16), jnp.float32),
            pltpu.VMEM((CH, D), jnp.float32),
            pltpu.VMEM((CH, ACCW), jnp.float32),
            pltpu.VMEM((CH,), jnp.int32),
            pltpu.VMEM((CH,), jnp.int32),
            pltpu.VMEM((CH + 16,), jnp.float32),
            pltpu.VMEM((CH,), jnp.int32),
            pltpu.VMEM((CH, 16), jnp.float32),
            pltpu.VMEM((CH, 16), jnp.float32),
            pltpu.VMEM((CH, D), jnp.float32),
            pltpu.VMEM((CH, ACCW), jnp.float32),
            pltpu.VMEM_SHARED((NPAD, ACCW), jnp.float32),
            pltpu.SemaphoreType.DMA,
            pltpu.SemaphoreType.DMA,
            pltpu.SemaphoreType.DMA,
            pltpu.SemaphoreType.DMA,
            pltpu.SemaphoreType.DMA,
            pltpu.SemaphoreType.DMA,
        ],
        compiler_params=pltpu.CompilerParams(use_tc_tiling_on_sc=False,
                                             internal_scratch_in_bytes=2 ** 17),
    )


def _feat_ext(x, w, al, ar):
    feat = lax.dot_general(x, w, (((1,), (1,)), ((), ())),
                           preferred_element_type=jnp.float32)
    el = jnp.dot(feat, al, preferred_element_type=jnp.float32)
    er = jnp.dot(feat, ar, preferred_element_type=jnp.float32)
    return feat, el, er


def _feat_body(x_ref, w_ref, al_ref, ar_ref, f_ref, el_ref, er_ref):
    feat, el, er = _feat_ext(x_ref[...], w_ref[...], al_ref[...], ar_ref[...])
    f_ref[...] = feat
    el_ref[...] = el
    er_ref[...] = er


_feat_call = pl.pallas_call(
    _feat_body,
    grid=(N // BN,),
    in_specs=[pl.BlockSpec((BN, D), lambda i: (i, 0)),
              pl.BlockSpec((D, D), lambda i: (0, 0)),
              pl.BlockSpec((D, 16), lambda i: (0, 0)),
              pl.BlockSpec((D, 16), lambda i: (0, 0))],
    out_specs=[pl.BlockSpec((BN, D), lambda i: (i, 0)),
               pl.BlockSpec((BN, 16), lambda i: (i, 0)),
               pl.BlockSpec((BN, 16), lambda i: (i, 0))],
    out_shape=[jax.ShapeDtypeStruct((N, D), jnp.float32),
               jax.ShapeDtypeStruct((N, 16), jnp.float32),
               jax.ShapeDtypeStruct((N, 16), jnp.float32)],
)


def _combine_rst(n0, n1, d0, d1, r, bd, b):
    a = n0 + n1
    den = jnp.dot((d0 + d1)[:, :H], r, preferred_element_type=jnp.float32)
    rst = jnp.where(den > 0, a / den, 0.0)
    y = jnp.dot(rst, bd, preferred_element_type=jnp.float32) + b
    return jnp.maximum(y, 0.0)


def _combine_feat_body(n0_ref, n1_ref, d0_ref, d1_ref, r_ref, bd_ref, b_ref,
                       w_ref, al_ref, ar_ref, x1_ref, f_ref, el_ref, er_ref):
    x1 = _combine_rst(n0_ref[...], n1_ref[...], d0_ref[...], d1_ref[...],
                      r_ref[...], bd_ref[...], b_ref[...])
    x1_ref[...] = x1
    feat, el, er = _feat_ext(x1, w_ref[...], al_ref[...], ar_ref[...])
    f_ref[...] = feat
    el_ref[...] = el
    er_ref[...] = er


_combine_feat_call = pl.pallas_call(
    _combine_feat_body,
    grid=(N // BN,),
    in_specs=[pl.BlockSpec((BN, D), lambda i: (i, 0)),
              pl.BlockSpec((BN, D), lambda i: (i, 0)),
              pl.BlockSpec((BN, 16), lambda i: (i, 0)),
              pl.BlockSpec((BN, 16), lambda i: (i, 0)),
              pl.BlockSpec((H, D), lambda i: (0, 0)),
              pl.BlockSpec((D, D), lambda i: (0, 0)),
              pl.BlockSpec((1, D), lambda i: (0, 0)),
              pl.BlockSpec((D, D), lambda i: (0, 0)),
              pl.BlockSpec((D, 16), lambda i: (0, 0)),
              pl.BlockSpec((D, 16), lambda i: (0, 0))],
    out_specs=[pl.BlockSpec((BN, D), lambda i: (i, 0)),
               pl.BlockSpec((BN, D), lambda i: (i, 0)),
               pl.BlockSpec((BN, 16), lambda i: (i, 0)),
               pl.BlockSpec((BN, 16), lambda i: (i, 0))],
    out_shape=[jax.ShapeDtypeStruct((N, D), jnp.float32),
               jax.ShapeDtypeStruct((N, D), jnp.float32),
               jax.ShapeDtypeStruct((N, 16), jnp.float32),
               jax.ShapeDtypeStruct((N, 16), jnp.float32)],
)


def _final_body(n0_ref, n1_ref, d0_ref, d1_ref, r_ref, bd_ref, b_ref,
                x1_ref, w0_ref, w1_ref, mb_ref, o_ref):
    out1 = _combine_rst(n0_ref[...], n1_ref[...], d0_ref[...], d1_ref[...],
                        r_ref[...], bd_ref[...], b_ref[...])
    o = (jnp.dot(x1_ref[...], w0_ref[...], preferred_element_type=jnp.float32)
         + jnp.dot(out1, w1_ref[...], preferred_element_type=jnp.float32)
         + mb_ref[...])
    o_ref[...] = o


_final_call = pl.pallas_call(
    _final_body,
    grid=(N // BN,),
    in_specs=[pl.BlockSpec((BN, D), lambda i: (i, 0)),
              pl.BlockSpec((BN, D), lambda i: (i, 0)),
              pl.BlockSpec((BN, 16), lambda i: (i, 0)),
              pl.BlockSpec((BN, 16), lambda i: (i, 0)),
              pl.BlockSpec((H, D), lambda i: (0, 0)),
              pl.BlockSpec((D, D), lambda i: (0, 0)),
              pl.BlockSpec((1, D), lambda i: (0, 0)),
              pl.BlockSpec((BN, D), lambda i: (i, 0)),
              pl.BlockSpec((D, D), lambda i: (0, 0)),
              pl.BlockSpec((D, D), lambda i: (0, 0)),
              pl.BlockSpec((1, D), lambda i: (0, 0))],
    out_specs=pl.BlockSpec((BN, D), lambda i: (i, 0)),
    out_shape=jax.ShapeDtypeStruct((N, D), jnp.float32),
)


def _attn_mats(attn_l, attn_r):
    m = jnp.kron(jnp.eye(H, dtype=jnp.float32),
                 jnp.ones((DH, 1), jnp.float32))
    al = attn_l.reshape(H * DH)[:, None] * m
    ar = attn_r.reshape(H * DH)[:, None] * m
    return (jnp.concatenate([al, al], axis=1),
            jnp.concatenate([ar, ar], axis=1))


def kernel(solutions, edge_index, edge_w, fc_W0, attn_l0, attn_r0, out_W0,
           out_b0, fc_W1, attn_l1, attn_r1, out_W1, out_b1, mlp_W, mlp_b):
    src = edge_index[0]
    dst = edge_index[1]
    x0 = jnp.concatenate([solutions[0], solutions[1]], axis=-1)[0]

    r_mat = jnp.kron(jnp.eye(H, dtype=jnp.float32),
                     jnp.ones((1, DH), jnp.float32))
    bd0 = jnp.kron(jnp.eye(H, dtype=jnp.float32), out_W0.T)
    bd1 = jnp.kron(jnp.eye(H, dtype=jnp.float32), out_W1.T)
    b0t = jnp.tile(out_b0, H)[None, :]
    b1t = jnp.tile(out_b1, H)[None, :]
    w0t = mlp_W[:, :D].T
    w1t = mlp_W[:, D:].T
    mbt = mlp_b[None, :]

    al0, ar0 = _attn_mats(attn_l0, attn_r0)
    al1, ar1 = _attn_mats(attn_l1, attn_r1)
    feat0, el0, er0 = _feat_call(x0, fc_W0, al0, ar0)
    num0, den0 = _edge_call()(feat0, el0, er0, src, dst, edge_w)
    x1, feat1, el1, er1 = _combine_feat_call(
        num0[:N], num0[NPAD:NPAD + N], den0[:N], den0[NPAD:NPAD + N],
        r_mat, bd0, b0t, fc_W1, al1, ar1)
    num1, den1 = _edge_call()(feat1, el1, er1, src, dst, edge_w)
    h = _final_call(num1[:N], num1[NPAD:NPAD + N], den1[:N],
                    den1[NPAD:NPAD + N], r_mat, bd1, b1t, x1, w0t, w1t, mbt)
    return h[None]
